# Initial kernel scaffold; baseline (speedup 1.0000x reference)
import jax
import jax.numpy as jnp
from jax.experimental import pallas as pl


def kernel(x, edge_index, edge_attr, l0_Wl, l0_bl, l0_Wr, l0_br, l0_We, l0_att, l0_bias, l0_ln_g, l0_ln_b, l1_Wl, l1_bl, l1_Wr, l1_br, l1_We, l1_att, l1_bias, l1_ln_g, l1_ln_b, p_ln_g, p_ln_b, p_W1, p_b1, p_bn1_g, p_bn1_b, p_W2, p_b2, p_bn2_g, p_bn2_b, p_W3, p_b3):
    raise NotImplementedError("write your pallas kernel here")


# speedup vs baseline: 9.8567x; 9.8567x over previous
import functools

import jax
import jax.numpy as jnp
from jax import lax
from jax.experimental import pallas as pl
from jax.experimental.pallas import tpu as pltpu
from jax.experimental.pallas import tpu_sc as plsc

N = 10000
E = 320000
D = 128
C = 128

NC = 2
NS = 16
NP = 10240
NSL = NP // NS
EPT = E // (NC * NS)
SB = 80
NBLK = EPT // SB
CHK = 2000
P0T = E // NS
EPS = 1e-8

_f32 = jnp.float32


def _sc_body(src_hbm, dst_hbm, ea_hbm, xl_hbm, xr_hbm, nl_hbm, nr_hbm,
             we_hbm, att_hbm,
             outp_hbm, sp_hbm,
             tbl_v, mxp, srcb, dstb, accb, tmpb,
             src_v, dst_v, ea_v, xl_rows, xr_rows, e_buf,
             ktmp, vtmp,
             we_v, att_v,
             out_sh, s_sh, msh2, denm, semA, semB):
    c = lax.axis_index("c")
    s = lax.axis_index("s")
    wid = c * NS + s
    z16 = jnp.zeros((16,), _f32)
    iota = jnp.arange(16, dtype=jnp.int32)
    base_n = s * NSL
    base_o = s * (N // NS)

    pltpu.sync_copy(nl_hbm, tbl_v)
    pltpu.sync_copy(we_hbm, we_v)
    pltpu.sync_copy(att_hbm, att_v)

    def zmx(i, carry):
        mxp[pl.ds(i * 16, 16)] = z16
        return carry
    lax.fori_loop(0, NP // 16, zmx, 0)

    def zacc(i, carry):
        accb[pl.ds(i * 16, 16)] = z16
        return carry
    lax.fori_loop(0, NSL // 16, zacc, 0)
    pltpu.sync_copy(accb, s_sh.at[pl.ds(base_n, NSL)])

    def zrows(i, carry):
        for ch in range(8):
            xl_rows[i, pl.ds(ch * 16, 16)] = z16
        return carry
    lax.fori_loop(0, SB, zrows, 0)
    woff = s * 624

    @pl.when(s < NS - 1)
    def _zero_624():
        for kk in range(7):
            pltpu.sync_copy(xl_rows, out_sh.at[pl.ds(woff + kk * SB, SB), :])
        pltpu.sync_copy(xl_rows.at[pl.ds(0, 64), :],
                        out_sh.at[pl.ds(woff + 560, 64), :])

    @pl.when(s == NS - 1)
    def _zero_640():
        for kk in range(8):
            pltpu.sync_copy(xl_rows, out_sh.at[pl.ds(9360 + kk * SB, SB), :])

    def p0chunk(kk, carry):
        off = s * P0T + kk * CHK
        pltpu.sync_copy(src_hbm.at[pl.ds(off, CHK)], srcb)
        pltpu.sync_copy(dst_hbm.at[pl.ds(off, CHK)], dstb)

        def p0in(i, carry2):
            b = i * 16
            sv = srcb[pl.ds(b, 16)]
            dv = dstb[pl.ds(b, 16)]
            nj = plsc.load_gather(tbl_v, [sv])
            dk, vals = plsc.sort_key_val(dv, nj)
            ktmp[...] = dk
            for o in (1, 2, 4, 8):
                vtmp[...] = vals
                sh = jnp.maximum(iota - o, 0)
                kp = plsc.load_gather(ktmp, [sh])
                vp = plsc.load_gather(vtmp, [sh])
                take = (kp == dk) & (iota >= o)
                vals = jnp.where(take, jnp.maximum(vals, vp), vals)
            knext = plsc.load_gather(ktmp, [jnp.minimum(iota + 1, 15)])
            last = (dk != knext) | (iota == 15)
            cur = plsc.load_gather(mxp, [dk])
            plsc.store_scatter(mxp, [dk], jnp.maximum(cur, vals), mask=last)
            return carry2
        lax.fori_loop(0, CHK // 16, p0in, 0)
        return carry
    lax.fori_loop(0, P0T // CHK, p0chunk, 0)

    def zacc2(i, carry):
        accb[pl.ds(i * 16, 16)] = z16
        return carry
    lax.fori_loop(0, NSL // 16, zacc2, 0)
    for r in range(NS):
        seg = lax.rem(s + r, NS)
        pltpu.sync_copy(mxp.at[pl.ds(seg * NSL, NSL)], msh2.at[s])
        plsc.subcore_barrier()
        t = lax.rem(s - r + NS, NS)
        pltpu.sync_copy(msh2.at[t], tmpb)

        def mrg(i, carry):
            sl = pl.ds(i * 16, 16)
            accb[sl] = jnp.maximum(accb[sl], tmpb[sl])
            return carry
        lax.fori_loop(0, NSL // 16, mrg, 0)
        plsc.subcore_barrier()

    pltpu.sync_copy(nr_hbm.at[pl.ds(base_n, NSL)], tmpb)

    def den_slice(i, carry):
        sl = pl.ds(i * 16, 16)
        accb[sl] = 2.0 * (tmpb[sl] + accb[sl] + 3.0 * EPS) + EPS
        return carry
    lax.fori_loop(0, NSL // 16, den_slice, 0)
    pltpu.sync_copy(accb, denm.at[pl.ds(base_n, NSL)])
    plsc.subcore_barrier()
    pltpu.sync_copy(denm, tbl_v)

    ebase = wid * EPT

    def sweep(k, carry):
        bb = ebase + k * SB
        pltpu.sync_copy(src_hbm.at[pl.ds(bb, SB)], src_v)
        pltpu.sync_copy(dst_hbm.at[pl.ds(bb, SB)], dst_v)
        pltpu.sync_copy(ea_hbm.at[pl.ds(bb * 4, SB * 4)],
                        ea_v.at[pl.ds(0, SB * 4)])
        cpa = pltpu.async_copy(xl_hbm.at[src_v], xl_rows, semA)
        cpb = pltpu.async_copy(xr_hbm.at[dst_v], xr_rows, semB)
        cpa.wait()
        cpb.wait()

        def group(i, carry2):
            b16 = i * 16
            dv = dst_v[pl.ds(b16, 16)]
            den16 = plsc.load_gather(tbl_v, [dv])
            esums = z16
            for u in range(16):
                j = b16 + u
                av = ea_v[pl.ds(4 * j, 16)]
                acc = z16
                for ch in range(8):
                    sl = pl.ds(ch * 16, 16)
                    t = (xr_rows[j, sl] + xl_rows[j, sl]
                         + av[0] * we_v[0, sl] + av[1] * we_v[1, sl]
                         + av[2] * we_v[2, sl] + av[3] * we_v[3, sl])
                    t = jnp.where(t >= 0.0, t, 0.01 * t)
                    acc = acc + t * att_v[sl]
                esums = jnp.where(iota == u, jnp.sum(acc), esums)
            ev = esums / den16
            ev = jnp.minimum(jnp.maximum(ev, -8.0), 8.0)
            ee16 = jnp.exp(ev)
            e_buf[pl.ds(b16, 16)] = ee16
            for u in range(16):
                j = b16 + u
                eej = ee16[u]
                for ch in range(8):
                    sl = pl.ds(ch * 16, 16)
                    xl_rows[j, sl] = xl_rows[j, sl] * eej
            return carry2
        lax.fori_loop(0, SB // 16, group, 0)

        pltpu.sync_copy(xl_rows, out_sh.at[dst_v], add=True)
        pltpu.sync_copy(e_buf, s_sh.at[dst_v], add=True)
        return carry
    lax.fori_loop(0, NBLK, sweep, 0)

    plsc.subcore_barrier()

    @pl.when(s < NS - 1)
    def _wr_624():
        pltpu.sync_copy(out_sh.at[pl.ds(woff, 624), :],
                        outp_hbm.at[c, pl.ds(woff, 624), :])

    @pl.when(s == NS - 1)
    def _wr_640():
        pltpu.sync_copy(out_sh.at[pl.ds(9360, 640), :],
                        outp_hbm.at[c, pl.ds(9360, 640), :])

    pltpu.sync_copy(s_sh.at[pl.ds(base_n, NSL)],
                    sp_hbm.at[c, pl.ds(base_n, NSL)])


def _gat_sc(src, dst, ea, xl, xr, nl, nr, we, att):
    mesh = plsc.VectorSubcoreMesh(core_axis_name="c", subcore_axis_name="s",
                                  num_cores=NC, num_subcores=NS)
    kfn = pl.kernel(
        _sc_body,
        out_type=[jax.ShapeDtypeStruct((NC, N, 128), _f32),
                  jax.ShapeDtypeStruct((NC, NP), _f32)],
        mesh=mesh,
        compiler_params=pltpu.CompilerParams(needs_layout_passes=False),
        scratch_types=[
            pltpu.VMEM((NP,), _f32),
            pltpu.VMEM((NP,), _f32),
            pltpu.VMEM((CHK,), jnp.int32),
            pltpu.VMEM((CHK,), jnp.int32),
            pltpu.VMEM((NSL,), _f32),
            pltpu.VMEM((NSL,), _f32),
            pltpu.VMEM((SB,), jnp.int32),
            pltpu.VMEM((SB,), jnp.int32),
            pltpu.VMEM((SB * 4 + 16,), _f32),
            pltpu.VMEM((SB, 128), _f32),
            pltpu.VMEM((SB, 128), _f32),
            pltpu.VMEM((SB,), _f32),
            pltpu.VMEM((16,), jnp.int32),
            pltpu.VMEM((16,), _f32),
            pltpu.VMEM((4, 128), _f32),
            pltpu.VMEM((128,), _f32),
            pltpu.VMEM_SHARED((N, 128), _f32),
            pltpu.VMEM_SHARED((NP,), _f32),
            pltpu.VMEM_SHARED((NS, NSL), _f32),
            pltpu.VMEM_SHARED((NP,), _f32),
            pltpu.SemaphoreType.DMA,
            pltpu.SemaphoreType.DMA,
        ],
    )
    nl_p = jnp.pad(nl, (0, NP - N))
    nr_p = jnp.pad(nr, (0, NP - N))
    return kfn(src, dst, ea, xl, xr, nl_p, nr_p, we, att)


RB = 1000
_BN_SCALE = 0.9999950000374997


def _lrelu(x):
    return jnp.where(x >= 0, x, 0.01 * x)


def _ln(x, g, b):
    m = jnp.mean(x, axis=1, keepdims=True)
    v = jnp.mean((x - m) * (x - m), axis=1, keepdims=True)
    return (x - m) / jnp.sqrt(v + 1e-5) * g + b


def _proj_body(x_ref, wl_ref, bl_ref, wr_ref, br_ref,
               xl_ref, xr_ref, nl_ref, nr_ref):
    xb = x_ref[...]
    xl = jnp.dot(xb, wl_ref[...], preferred_element_type=_f32) + bl_ref[...]
    xr = jnp.dot(xb, wr_ref[...], preferred_element_type=_f32) + br_ref[...]
    xl_ref[...] = xl
    xr_ref[...] = xr
    nl_ref[...] = jnp.sqrt(jnp.sum(xl * xl, axis=1, keepdims=True))
    nr_ref[...] = jnp.sqrt(jnp.sum(xr * xr, axis=1, keepdims=True))


def _proj(x, wl, bl, wr, br):
    row = lambda i: (i, 0)
    full = lambda i: (0, 0)
    return pl.pallas_call(
        _proj_body,
        grid=(N // RB,),
        in_specs=[
            pl.BlockSpec((RB, D), row),
            pl.BlockSpec((D, C), full),
            pl.BlockSpec((1, C), full),
            pl.BlockSpec((D, C), full),
            pl.BlockSpec((1, C), full),
        ],
        out_specs=[
            pl.BlockSpec((RB, C), row),
            pl.BlockSpec((RB, C), row),
            pl.BlockSpec((RB, 1), row),
            pl.BlockSpec((RB, 1), row),
        ],
        out_shape=[
            jax.ShapeDtypeStruct((N, C), _f32),
            jax.ShapeDtypeStruct((N, C), _f32),
            jax.ShapeDtypeStruct((N, 1), _f32),
            jax.ShapeDtypeStruct((N, 1), _f32),
        ],
    )(x, wl, bl, wr, br)


def _mid_body(o0_ref, o1_ref, s0_ref, s1_ref, bias_ref, g0_ref, b0_ref,
              wl_ref, bl_ref, wr_ref, br_ref,
              h_ref, xl_ref, xr_ref, nl_ref, nr_ref):
    ssum = s0_ref[...] + s1_ref[...] + 1e-16
    g = (o0_ref[...] + o1_ref[...]) / ssum + bias_ref[...]
    h = _lrelu(_ln(g, g0_ref[...], b0_ref[...]))
    h_ref[...] = h
    xl = jnp.dot(h, wl_ref[...], preferred_element_type=_f32) + bl_ref[...]
    xr = jnp.dot(h, wr_ref[...], preferred_element_type=_f32) + br_ref[...]
    xl_ref[...] = xl
    xr_ref[...] = xr
    nl_ref[...] = jnp.sqrt(jnp.sum(xl * xl, axis=1, keepdims=True))
    nr_ref[...] = jnp.sqrt(jnp.sum(xr * xr, axis=1, keepdims=True))


def _mid(o0, o1, s0, s1, bias, g0, b0, wl, bl, wr, br):
    row = lambda i: (i, 0)
    full = lambda i: (0, 0)
    return pl.pallas_call(
        _mid_body,
        grid=(N // RB,),
        in_specs=[
            pl.BlockSpec((RB, C), row),
            pl.BlockSpec((RB, C), row),
            pl.BlockSpec((RB, 1), row),
            pl.BlockSpec((RB, 1), row),
            pl.BlockSpec((1, C), full),
            pl.BlockSpec((1, C), full),
            pl.BlockSpec((1, C), full),
            pl.BlockSpec((D, C), full),
            pl.BlockSpec((1, C), full),
            pl.BlockSpec((D, C), full),
            pl.BlockSpec((1, C), full),
        ],
        out_specs=[
            pl.BlockSpec((RB, C), row),
            pl.BlockSpec((RB, C), row),
            pl.BlockSpec((RB, C), row),
            pl.BlockSpec((RB, 1), row),
            pl.BlockSpec((RB, 1), row),
        ],
        out_shape=[
            jax.ShapeDtypeStruct((N, C), _f32),
            jax.ShapeDtypeStruct((N, C), _f32),
            jax.ShapeDtypeStruct((N, C), _f32),
            jax.ShapeDtypeStruct((N, 1), _f32),
            jax.ShapeDtypeStruct((N, 1), _f32),
        ],
    )(o0, o1, s0, s1, bias, g0, b0, wl, bl, wr, br)


def _final_body(o0_ref, o1_ref, s0_ref, s1_ref, res_ref, bias_ref,
                g1_ref, b1_ref, pg_ref, pb_ref,
                w1_ref, bw1_ref, bn1g_ref, bn1b_ref,
                w2_ref, bw2_ref, bn2g_ref, bn2b_ref,
                w3_ref, bw3_ref, out_ref):
    ssum = s0_ref[...] + s1_ref[...] + 1e-16
    g = (o0_ref[...] + o1_ref[...]) / ssum + bias_ref[...]
    h2 = _ln(g, g1_ref[...], b1_ref[...]) + 0.1 * res_ref[...]
    h2 = _lrelu(h2)
    z = _ln(h2, pg_ref[...], pb_ref[...])
    z = jnp.dot(z, w1_ref[...], preferred_element_type=_f32) + bw1_ref[...]
    z = _lrelu(z * _BN_SCALE * bn1g_ref[...] + bn1b_ref[...])
    z = jnp.dot(z, w2_ref[...], preferred_element_type=_f32) + bw2_ref[...]
    z = _lrelu(z * _BN_SCALE * bn2g_ref[...] + bn2b_ref[...])
    out_ref[...] = (jnp.dot(z, w3_ref[...], preferred_element_type=_f32)
                    + bw3_ref[...])


def _final(o0, o1, s0, s1, res, bias, g1, b1, pg, pb,
           w1, bw1, bn1g, bn1b, w2, bw2, bn2g, bn2b, w3, bw3):
    row = lambda i: (i, 0)
    full = lambda i: (0, 0)
    dd = w1.shape[1]
    dh = w2.shape[1]
    do = w3.shape[1]
    return pl.pallas_call(
        _final_body,
        grid=(N // RB,),
        in_specs=[
            pl.BlockSpec((RB, C), row),
            pl.BlockSpec((RB, C), row),
            pl.BlockSpec((RB, 1), row),
            pl.BlockSpec((RB, 1), row),
            pl.BlockSpec((RB, C), row),
            pl.BlockSpec((1, C), full),
            pl.BlockSpec((1, C), full),
            pl.BlockSpec((1, C), full),
            pl.BlockSpec((1, C), full),
            pl.BlockSpec((1, C), full),
            pl.BlockSpec((C, dd), full),
            pl.BlockSpec((1, dd), full),
            pl.BlockSpec((1, dd), full),
            pl.BlockSpec((1, dd), full),
            pl.BlockSpec((dd, dh), full),
            pl.BlockSpec((1, dh), full),
            pl.BlockSpec((1, dh), full),
            pl.BlockSpec((1, dh), full),
            pl.BlockSpec((dh, do), full),
            pl.BlockSpec((1, do), full),
        ],
        out_specs=pl.BlockSpec((RB, do), row),
        out_shape=jax.ShapeDtypeStruct((N, do), _f32),
    )(o0, o1, s0, s1, res, bias, g1, b1, pg, pb,
      w1, bw1, bn1g, bn1b, w2, bw2, bn2g, bn2b, w3, bw3)


def kernel(x, edge_index, edge_attr,
           l0_Wl, l0_bl, l0_Wr, l0_br, l0_We, l0_att, l0_bias, l0_ln_g, l0_ln_b,
           l1_Wl, l1_bl, l1_Wr, l1_br, l1_We, l1_att, l1_bias, l1_ln_g, l1_ln_b,
           p_ln_g, p_ln_b, p_W1, p_b1, p_bn1_g, p_bn1_b,
           p_W2, p_b2, p_bn2_g, p_bn2_b, p_W3, p_b3):
    src = edge_index[0]
    dst = edge_index[1]

    xl0, xr0, nl0, nr0 = _proj(x, l0_Wl, l0_bl[None], l0_Wr, l0_br[None])
    ea_flat = edge_attr.reshape(E * 4)
    outp0, sp0 = _gat_sc(src, dst, ea_flat, xl0, xr0,
                         nl0.reshape(N), nr0.reshape(N),
                         l0_We, l0_att.reshape(C))
    h, xl1, xr1, nl1, nr1 = _mid(
        outp0[0, :N], outp0[1, :N], sp0[0, :N, None], sp0[1, :N, None],
        l0_bias[None], l0_ln_g[None], l0_ln_b[None],
        l1_Wl, l1_bl[None], l1_Wr, l1_br[None])
    outp1, sp1 = _gat_sc(src, dst, ea_flat, xl1, xr1,
                         nl1.reshape(N), nr1.reshape(N),
                         l1_We, l1_att.reshape(C))
    out = _final(
        outp1[0, :N], outp1[1, :N], sp1[0, :N, None], sp1[1, :N, None],
        h, l1_bias[None], l1_ln_g[None], l1_ln_b[None],
        p_ln_g[None], p_ln_b[None],
        p_W1, p_b1[None], p_bn1_g[None], p_bn1_b[None],
        p_W2, p_b2[None], p_bn2_g[None], p_bn2_b[None],
        p_W3, p_b3[None])
    return out
